# baseline (speedup 1.0000x reference)
import functools

import jax
import jax.numpy as jnp
from jax import lax
from jax.experimental import pallas as pl
from jax.experimental.pallas import tpu as pltpu
from jax.experimental.pallas import tpu_sc as plsc

NC = 2
NS = 16
L = 16
NW = NC * NS

D = 64
B = 16384
BPW = B // NW

_MESH = plsc.VectorSubcoreMesh(core_axis_name="c", subcore_axis_name="s")


@functools.partial(
    pl.kernel,
    mesh=_MESH,
    out_type=jax.ShapeDtypeStruct((B, D), jnp.float32),
    scratch_types=[
        pltpu.VMEM((BPW,), jnp.int32),
        pltpu.VMEM((BPW, D), jnp.float32),
        pltpu.SemaphoreType.DMA,
    ],
    compiler_params=pltpu.CompilerParams(
        use_tc_tiling_on_sc=False,
        needs_layout_passes=False,
    ),
)
def _embed(table_hbm, labels_hbm, out_hbm, idx_v, rows_v, sem):
    wid = lax.axis_index("s") * NC + lax.axis_index("c")
    base = wid * BPW

    pltpu.sync_copy(labels_hbm.at[pl.ds(base, BPW)], idx_v)
    pltpu.async_copy(table_hbm.at[idx_v], rows_v, sem).wait()

    def body(b, carry):
        rvec = b * L + lax.iota(jnp.int32, L)
        acc = jnp.zeros((L,), jnp.float32)
        for j in range(D):
            v = plsc.load_gather(rows_v, [rvec, jnp.full((L,), j, jnp.int32)])
            acc = acc + v * v
        x = acc
        i = plsc.bitcast(x, jnp.int32)
        y = plsc.bitcast(jnp.int32(0x5F3759DF) - (i >> 1), jnp.float32)
        h = x * 0.5
        for _ in range(3):
            y = y * (1.5 - (h * y) * y)
        norm = x * y
        scale = jnp.minimum(1.0, 1.0 / (norm + 1e-7))
        for j in range(D):
            jv = jnp.full((L,), j, jnp.int32)
            col = plsc.load_gather(rows_v, [rvec, jv])
            plsc.store_scatter(rows_v, [rvec, jv], col * scale)
        return carry

    lax.fori_loop(0, BPW // L, body, 0)

    pltpu.sync_copy(rows_v, out_hbm.at[pl.ds(base, BPW)])


def kernel(table, labels):
    return _embed(table, labels.astype(jnp.int32))


# speedup vs baseline: 1.6062x; 1.6062x over previous
import functools

import jax
import jax.numpy as jnp
from jax import lax
from jax.experimental import pallas as pl
from jax.experimental.pallas import tpu as pltpu
from jax.experimental.pallas import tpu_sc as plsc

NC = 2
NS = 16
L = 16
NW = NC * NS

D = 64
B = 16384
BPW = B // NW

_MESH = plsc.VectorSubcoreMesh(core_axis_name="c", subcore_axis_name="s")


@functools.partial(
    pl.kernel,
    mesh=_MESH,
    out_type=jax.ShapeDtypeStruct((B, D), jnp.float32),
    scratch_types=[
        pltpu.VMEM((BPW,), jnp.int32),
        pltpu.VMEM((BPW, D), jnp.float32),
        pltpu.SemaphoreType.DMA,
    ],
    compiler_params=pltpu.CompilerParams(
        needs_layout_passes=False,
    ),
)
def _embed(table_hbm, labels_hbm, out_hbm, idx_v, rows_v, sem):
    wid = lax.axis_index("s") * NC + lax.axis_index("c")
    base = wid * BPW

    pltpu.sync_copy(labels_hbm.at[pl.ds(base, BPW)], idx_v)

    def fire_block(b):
        iv = idx_v[pl.ds(b * L, L)]
        for l in range(L):
            pltpu.async_copy(table_hbm.at[iv[l]], rows_v.at[b * L + l], sem)

    def drain_block(b):
        for l in range(L):
            pltpu.make_async_copy(table_hbm.at[0], rows_v.at[b * L + l], sem).wait()

    NBLK = BPW // L
    fire_block(0)

    def pipe_body(b, carry):
        fire_block(b)
        drain_block(b - 1)
        return carry

    lax.fori_loop(1, NBLK, pipe_body, 0)
    drain_block(NBLK - 1)

    def body(b, carry):
        rvec = b * L + lax.iota(jnp.int32, L)
        acc = jnp.zeros((L,), jnp.float32)
        for j in range(D):
            v = plsc.load_gather(rows_v, [rvec, jnp.full((L,), j, jnp.int32)])
            acc = acc + v * v
        x = acc
        i = plsc.bitcast(x, jnp.int32)
        y = plsc.bitcast(jnp.int32(0x5F3759DF) - (i >> 1), jnp.float32)
        h = x * 0.5
        for _ in range(3):
            y = y * (1.5 - (h * y) * y)
        norm = x * y
        scale = jnp.minimum(1.0, 1.0 / (norm + 1e-7))
        for j in range(D):
            jv = jnp.full((L,), j, jnp.int32)
            col = plsc.load_gather(rows_v, [rvec, jv])
            plsc.store_scatter(rows_v, [rvec, jv], col * scale)
        return carry

    lax.fori_loop(0, BPW // L, body, 0)

    pltpu.sync_copy(rows_v, out_hbm.at[pl.ds(base, BPW)])


def kernel(table, labels):
    return _embed(table, labels.astype(jnp.int32))


# speedup vs baseline: 2.1202x; 1.3200x over previous
import functools

import jax
import jax.numpy as jnp
from jax import lax
from jax.experimental import pallas as pl
from jax.experimental.pallas import tpu as pltpu
from jax.experimental.pallas import tpu_sc as plsc

NC = 2
NS = 16
L = 16
NW = NC * NS

V = 1000000
D = 64
B = 16384

TILE_COLS = (V + 127) // 128
BASE_TPW = TILE_COLS // NW
EXTRA = TILE_COLS - BASE_TPW * NW

RING = 32

_MESH = plsc.VectorSubcoreMesh(core_axis_name="c", subcore_axis_name="s")

_GDN = lax.GatherDimensionNumbers(
    offset_dims=(), collapsed_slice_dims=(0,), start_index_map=(0,)
)


def _lane_shuffle(v, idx):
    return lax.gather(
        v,
        idx[:, None],
        _GDN,
        (1,),
        mode=lax.GatherScatterMode.PROMISE_IN_BOUNDS,
    )


@functools.partial(
    pl.kernel,
    mesh=_MESH,
    out_type=jax.ShapeDtypeStruct((B * D,), jnp.float32),
    scratch_types=[
        pltpu.VMEM((B,), jnp.int32),
        pltpu.VMEM((B,), jnp.int32),
        pltpu.VMEM((B,), jnp.int32),
        pltpu.VMEM((L,), jnp.int32),
        pltpu.VMEM((L,), jnp.int32),
        pltpu.VMEM((2, D, 128), jnp.float32),
        pltpu.VMEM((RING, D), jnp.float32),
        pltpu.SemaphoreType.DMA,
        pltpu.SemaphoreType.DMA,
    ],
    compiler_params=pltpu.CompilerParams(
        needs_layout_passes=False,
        disable_bounds_checks=True,
    ),
)
def _embed(tableT_hbm, labels_hbm, out_hbm, lab_v, rlist, blist, s16r, s16b,
           tiles, ring, sem_in, sem_out):
    wid = lax.axis_index("s") * NC + lax.axis_index("c")
    jlo = wid * BASE_TPW + jnp.minimum(wid, EXTRA)
    ntiles = BASE_TPW + (wid < EXTRA).astype(jnp.int32)
    rlo = jlo * 128
    rhi = (jlo + ntiles) * 128

    iota = lax.iota(jnp.int32, L)

    pltpu.sync_copy(labels_hbm, lab_v)

    def fill_body(t, carry):
        rlist[pl.ds(t * L, L)] = jnp.full((L,), -1, jnp.int32)
        return carry

    lax.fori_loop(0, B // L, fill_body, 0)

    def filt_body(t, cnt):
        rv = lab_v[pl.ds(t * L, L)]
        m = (rv >= rlo) & (rv < rhi)
        nm = plsc.all_reduce_population_count(m)[0]
        plsc.store_compressed(s16r.at[...], rv, mask=m)
        plsc.store_compressed(s16b.at[...], t * L + iota, mask=m)
        dst = cnt + iota
        wm = iota < nm
        plsc.store_scatter(rlist, [dst], s16r[...], mask=wm)
        plsc.store_scatter(blist, [dst], s16b[...], mask=wm)
        return cnt + nm

    cnt = lax.fori_loop(0, B // L, filt_body, jnp.int32(0))
    tmax = (cnt + L - 1) // L

    def start_tile(cc):
        j = jnp.minimum(jlo + cc, TILE_COLS - 1)
        src = tableT_hbm.at[:, pl.ds(pl.multiple_of(j * 128, 128), 128)]
        pltpu.async_copy(src, tiles.at[cc & 1], sem_in)

    start_tile(jnp.int32(0))

    dummy_row = out_hbm.at[pl.ds(0, D)]

    def tile_body(cc, fired):
        pltpu.make_async_copy(
            tableT_hbm.at[:, pl.ds(0, 128)], tiles.at[cc & 1], sem_in
        ).wait()
        start_tile(cc + 1)
        j = jlo + cc
        buf = tiles.at[cc & 1]

        def scan_body(t, fired):
            rv = rlist[pl.ds(t * L, L)]
            m = (rv >> 7) == j
            nm = plsc.all_reduce_population_count(m)[0]
            plsc.store_compressed(s16r.at[...], rv, mask=m)
            plsc.store_compressed(s16b.at[...], blist[pl.ds(t * L, L)], mask=m)

            def lab_body(l, fired):
                lsel = jnp.full((L,), l, jnp.int32)
                r = plsc.load_gather(s16r, [lsel])[0]
                b = plsc.load_gather(s16b, [lsel])[0]
                cl = jnp.full((L,), r & 127, jnp.int32)
                v0 = plsc.load_gather(buf, [iota, cl])
                v1 = plsc.load_gather(buf, [iota + L, cl])
                v2 = plsc.load_gather(buf, [iota + 2 * L, cl])
                v3 = plsc.load_gather(buf, [iota + 3 * L, cl])
                acc = v0 * v0 + v1 * v1 + v2 * v2 + v3 * v3
                for sh in (8, 4, 2, 1):
                    acc = acc + _lane_shuffle(acc, iota ^ sh)
                x = acc
                i = plsc.bitcast(x, jnp.int32)
                y = plsc.bitcast(jnp.int32(0x5F3759DF) - (i >> 1), jnp.float32)
                h = x * 0.5
                for _ in range(3):
                    y = y * (1.5 - (h * y) * y)
                norm = x * y
                scale = jnp.minimum(1.0, 1.0 / (norm + 1e-7))
                slot = fired & (RING - 1)
                pl.when(fired >= RING)(
                    lambda: pltpu.make_async_copy(
                        dummy_row, ring.at[slot], sem_out
                    ).wait()
                )
                ring[slot, pl.ds(0, L)] = v0 * scale
                ring[slot, pl.ds(L, L)] = v1 * scale
                ring[slot, pl.ds(2 * L, L)] = v2 * scale
                ring[slot, pl.ds(3 * L, L)] = v3 * scale
                pltpu.async_copy(
                    ring.at[slot], out_hbm.at[pl.ds(b * D, D)], sem_out
                )
                return fired + 1

            return lax.fori_loop(0, nm, lab_body, fired)

        return lax.fori_loop(0, tmax, scan_body, fired)

    fired = lax.fori_loop(0, ntiles, tile_body, jnp.int32(0))

    pltpu.make_async_copy(
        tableT_hbm.at[:, pl.ds(0, 128)], tiles.at[ntiles & 1], sem_in
    ).wait()

    def drain_body(d, carry):
        pl.when(d < jnp.minimum(fired, RING))(
            lambda: pltpu.make_async_copy(
                dummy_row, ring.at[0], sem_out
            ).wait()
        )
        return carry

    lax.fori_loop(0, RING, drain_body, 0)


def kernel(table, labels):
    out = _embed(table.T, labels.astype(jnp.int32))
    return out.reshape(B, D)


# speedup vs baseline: 2.1946x; 1.0351x over previous
import functools

import jax
import jax.numpy as jnp
from jax import lax
from jax.experimental import pallas as pl
from jax.experimental.pallas import tpu as pltpu
from jax.experimental.pallas import tpu_sc as plsc

NC = 2
NS = 16
L = 16
NW = NC * NS

V = 1000000
D = 64
B = 16384

TILE_COLS = (V + 127) // 128
BASE_TPW = TILE_COLS // NW
EXTRA = TILE_COLS - BASE_TPW * NW

RING = 32
NBKT = 16

_MESH = plsc.VectorSubcoreMesh(core_axis_name="c", subcore_axis_name="s")

_GDN = lax.GatherDimensionNumbers(
    offset_dims=(), collapsed_slice_dims=(0,), start_index_map=(0,)
)


def _lane_shuffle(v, idx):
    return lax.gather(
        v,
        idx[:, None],
        _GDN,
        (1,),
        mode=lax.GatherScatterMode.PROMISE_IN_BOUNDS,
    )


@functools.partial(
    pl.kernel,
    mesh=_MESH,
    out_type=jax.ShapeDtypeStruct((B * D,), jnp.float32),
    scratch_types=[
        pltpu.VMEM((B,), jnp.int32),
        pltpu.VMEM((B,), jnp.int32),
        pltpu.VMEM((B,), jnp.int32),
        pltpu.VMEM((B,), jnp.int32),
        pltpu.VMEM((B,), jnp.int32),
        pltpu.SMEM((NBKT + 1,), jnp.int32),
        pltpu.VMEM((L,), jnp.int32),
        pltpu.VMEM((L,), jnp.int32),
        pltpu.VMEM((2, D, 128), jnp.float32),
        pltpu.VMEM((RING, D), jnp.float32),
        pltpu.SemaphoreType.DMA,
        pltpu.SemaphoreType.DMA,
    ],
    compiler_params=pltpu.CompilerParams(
        needs_layout_passes=False,
        disable_bounds_checks=True,
    ),
)
def _embed(tableT_hbm, labels_hbm, out_hbm, lab_v, rlist, blist, rlist2,
           blist2, seg, s16r, s16b, tiles, ring, sem_in, sem_out):
    wid = lax.axis_index("s") * NC + lax.axis_index("c")
    jlo = wid * BASE_TPW + jnp.minimum(wid, EXTRA)
    ntiles = BASE_TPW + (wid < EXTRA).astype(jnp.int32)
    rlo = jlo * 128
    rhi = (jlo + ntiles) * 128

    iota = lax.iota(jnp.int32, L)

    pltpu.sync_copy(labels_hbm, lab_v)

    def fill_body(t, carry):
        rlist[pl.ds(t * L, L)] = jnp.full((L,), -1, jnp.int32)
        rlist2[pl.ds(t * L, L)] = jnp.full((L,), -1, jnp.int32)
        return carry

    lax.fori_loop(0, B // L, fill_body, 0)

    def filt_body(t, cnt):
        rv = lab_v[pl.ds(t * L, L)]
        m = (rv >= rlo) & (rv < rhi)
        nm = plsc.all_reduce_population_count(m)[0]
        plsc.store_compressed(s16r.at[...], rv, mask=m)
        plsc.store_compressed(s16b.at[...], t * L + iota, mask=m)
        dst = cnt + iota
        wm = iota < nm
        plsc.store_scatter(rlist, [dst], s16r[...], mask=wm)
        plsc.store_scatter(blist, [dst], s16b[...], mask=wm)
        return cnt + nm

    cnt = lax.fori_loop(0, B // L, filt_body, jnp.int32(0))
    tmax = (cnt + L - 1) // L

    seg[0] = jnp.int32(0)
    cnt2 = jnp.int32(0)
    for bkt in range(NBKT):
        def bkt_body(t, c2, _bkt=bkt):
            rv = rlist[pl.ds(t * L, L)]
            m = (((rv >> 7) - jlo) >> 4) == _bkt
            nm = plsc.all_reduce_population_count(m)[0]
            plsc.store_compressed(s16r.at[...], rv, mask=m)
            plsc.store_compressed(s16b.at[...], blist[pl.ds(t * L, L)], mask=m)
            wm = iota < nm
            plsc.store_scatter(rlist2, [c2 + iota], s16r[...], mask=wm)
            plsc.store_scatter(blist2, [c2 + iota], s16b[...], mask=wm)
            return c2 + nm

        cnt2 = lax.fori_loop(0, tmax, bkt_body, cnt2)
        seg[bkt + 1] = cnt2

    def start_tile(cc):
        j = jnp.minimum(jlo + cc, TILE_COLS - 1)
        src = tableT_hbm.at[:, pl.ds(pl.multiple_of(j * 128, 128), 128)]
        pltpu.async_copy(src, tiles.at[cc & 1], sem_in)

    start_tile(jnp.int32(0))

    dummy_row = out_hbm.at[pl.ds(0, D)]

    def tile_body(cc, fired):
        pltpu.make_async_copy(
            tableT_hbm.at[:, pl.ds(0, 128)], tiles.at[cc & 1], sem_in
        ).wait()
        start_tile(cc + 1)
        j = jlo + cc
        buf = tiles.at[cc & 1]

        bkt = cc >> 4
        t0 = seg[bkt] >> 4
        t1 = (seg[bkt + 1] + L - 1) >> 4

        def scan_body(t, fired):
            rv = rlist2[pl.ds(t * L, L)]
            m = (rv >> 7) == j
            nm = plsc.all_reduce_population_count(m)[0]
            plsc.store_compressed(s16r.at[...], rv, mask=m)
            plsc.store_compressed(s16b.at[...], blist2[pl.ds(t * L, L)], mask=m)

            def lab_body(l, fired):
                lsel = jnp.full((L,), l, jnp.int32)
                r = plsc.load_gather(s16r, [lsel])[0]
                b = plsc.load_gather(s16b, [lsel])[0]
                cl = jnp.full((L,), r & 127, jnp.int32)
                v0 = plsc.load_gather(buf, [iota, cl])
                v1 = plsc.load_gather(buf, [iota + L, cl])
                v2 = plsc.load_gather(buf, [iota + 2 * L, cl])
                v3 = plsc.load_gather(buf, [iota + 3 * L, cl])
                acc = v0 * v0 + v1 * v1 + v2 * v2 + v3 * v3
                for sh in (8, 4, 2, 1):
                    acc = acc + _lane_shuffle(acc, iota ^ sh)
                x = acc
                i = plsc.bitcast(x, jnp.int32)
                y = plsc.bitcast(jnp.int32(0x5F3759DF) - (i >> 1), jnp.float32)
                h = x * 0.5
                for _ in range(3):
                    y = y * (1.5 - (h * y) * y)
                norm = x * y
                scale = jnp.minimum(1.0, 1.0 / (norm + 1e-7))
                slot = fired & (RING - 1)
                pl.when(fired >= RING)(
                    lambda: pltpu.make_async_copy(
                        dummy_row, ring.at[slot], sem_out
                    ).wait()
                )
                ring[slot, pl.ds(0, L)] = v0 * scale
                ring[slot, pl.ds(L, L)] = v1 * scale
                ring[slot, pl.ds(2 * L, L)] = v2 * scale
                ring[slot, pl.ds(3 * L, L)] = v3 * scale
                pltpu.async_copy(
                    ring.at[slot], out_hbm.at[pl.ds(b * D, D)], sem_out
                )
                return fired + 1

            return lax.fori_loop(0, nm, lab_body, fired)

        return lax.fori_loop(t0, t1, scan_body, fired)

    fired = lax.fori_loop(0, ntiles, tile_body, jnp.int32(0))

    pltpu.make_async_copy(
        tableT_hbm.at[:, pl.ds(0, 128)], tiles.at[ntiles & 1], sem_in
    ).wait()

    def drain_body(d, carry):
        pl.when(d < jnp.minimum(fired, RING))(
            lambda: pltpu.make_async_copy(
                dummy_row, ring.at[0], sem_out
            ).wait()
        )
        return carry

    lax.fori_loop(0, RING, drain_body, 0)


def kernel(table, labels):
    out = _embed(table.T, labels.astype(jnp.int32))
    return out.reshape(B, D)


# speedup vs baseline: 3.9546x; 1.8020x over previous
import functools

import jax
import jax.numpy as jnp
from jax import lax
from jax.experimental import pallas as pl
from jax.experimental.pallas import tpu as pltpu
from jax.experimental.pallas import tpu_sc as plsc

NC = 2
NS = 16
L = 16
NW = NC * NS

V = 1000000
D = 64
B = 16384

TILE_COLS = (V + 127) // 128
BASE_TPW = TILE_COLS // NW
EXTRA = TILE_COLS - BASE_TPW * NW

RING = 32
NBKT = 16
NBUF = 4

_MESH = plsc.VectorSubcoreMesh(core_axis_name="c", subcore_axis_name="s")

_GDN = lax.GatherDimensionNumbers(
    offset_dims=(), collapsed_slice_dims=(0,), start_index_map=(0,)
)


def _lane_shuffle(v, idx):
    return lax.gather(
        v,
        idx[:, None],
        _GDN,
        (1,),
        mode=lax.GatherScatterMode.PROMISE_IN_BOUNDS,
    )


@functools.partial(
    pl.kernel,
    mesh=_MESH,
    out_type=jax.ShapeDtypeStruct((B * D,), jnp.float32),
    scratch_types=[
        pltpu.VMEM((B,), jnp.int32),
        pltpu.VMEM((B,), jnp.int32),
        pltpu.VMEM((B,), jnp.int32),
        pltpu.VMEM((B,), jnp.int32),
        pltpu.VMEM((B,), jnp.int32),
        pltpu.SMEM((NBKT + 1,), jnp.int32),
        pltpu.VMEM((L,), jnp.int32),
        pltpu.VMEM((L,), jnp.int32),
        pltpu.VMEM((NBUF, D, 128), jnp.float32),
        pltpu.VMEM((RING, D), jnp.float32),
        pltpu.SemaphoreType.DMA,
        pltpu.SemaphoreType.DMA,
    ],
    compiler_params=pltpu.CompilerParams(
        needs_layout_passes=False,
        disable_bounds_checks=True,
    ),
)
def _embed(tableT_hbm, labels_hbm, out_hbm, lab_v, rlist, blist, rlist2,
           blist2, seg, s16r, s16b, tiles, ring, sem_in, sem_out):
    wid = lax.axis_index("s") * NC + lax.axis_index("c")
    jlo = wid * BASE_TPW + jnp.minimum(wid, EXTRA)
    ntiles = BASE_TPW + (wid < EXTRA).astype(jnp.int32)
    rlo = jlo * 128
    rhi = (jlo + ntiles) * 128

    iota = lax.iota(jnp.int32, L)

    pltpu.sync_copy(labels_hbm, lab_v)

    def fill_body(t, carry):
        rlist[pl.ds(t * L, L)] = jnp.full((L,), -1, jnp.int32)
        rlist2[pl.ds(t * L, L)] = jnp.full((L,), -1, jnp.int32)
        return carry

    lax.fori_loop(0, B // L, fill_body, 0)

    def filt_body(t, cnt):
        rv = lab_v[pl.ds(t * L, L)]
        m = (rv >= rlo) & (rv < rhi)
        nm = plsc.all_reduce_population_count(m)[0]
        plsc.store_compressed(s16r.at[...], rv, mask=m)
        plsc.store_compressed(s16b.at[...], t * L + iota, mask=m)
        dst = cnt + iota
        wm = iota < nm
        plsc.store_scatter(rlist, [dst], s16r[...], mask=wm)
        plsc.store_scatter(blist, [dst], s16b[...], mask=wm)
        return cnt + nm

    cnt = lax.fori_loop(0, B // L, filt_body, jnp.int32(0))
    tmax = (cnt + L - 1) // L

    seg[0] = jnp.int32(0)
    cnt2 = jnp.int32(0)
    for bkt in range(NBKT):
        def bkt_body(t, c2, _bkt=bkt):
            rv = rlist[pl.ds(t * L, L)]
            m = (((rv >> 7) - jlo) >> 4) == _bkt
            nm = plsc.all_reduce_population_count(m)[0]
            plsc.store_compressed(s16r.at[...], rv, mask=m)
            plsc.store_compressed(s16b.at[...], blist[pl.ds(t * L, L)], mask=m)
            wm = iota < nm
            plsc.store_scatter(rlist2, [c2 + iota], s16r[...], mask=wm)
            plsc.store_scatter(blist2, [c2 + iota], s16b[...], mask=wm)
            return c2 + nm

        cnt2 = lax.fori_loop(0, tmax, bkt_body, cnt2)
        seg[bkt + 1] = cnt2

    def start_tile(cc):
        j = jnp.minimum(jlo + cc, TILE_COLS - 1)
        src = tableT_hbm.at[:, pl.ds(pl.multiple_of(j * 128, 128), 128)]
        pltpu.async_copy(src, tiles.at[cc & (NBUF - 1)], sem_in)

    for p in range(NBUF - 1):
        start_tile(jnp.int32(p))

    dummy_row = out_hbm.at[pl.ds(0, D)]

    def tile_body(cc, fired):
        pltpu.make_async_copy(
            tableT_hbm.at[:, pl.ds(0, 128)], tiles.at[cc & (NBUF - 1)], sem_in
        ).wait()
        start_tile(cc + NBUF - 1)
        j = jlo + cc
        buf = tiles.at[cc & (NBUF - 1)]

        bkt = cc >> 4
        t0 = seg[bkt] >> 4
        t1 = (seg[bkt + 1] + L - 1) >> 4

        def scan_body(t, fired):
            rv = rlist2[pl.ds(t * L, L)]
            m = (rv >> 7) == j
            nm = plsc.all_reduce_population_count(m)[0]
            plsc.store_compressed(s16r.at[...], rv, mask=m)
            plsc.store_compressed(s16b.at[...], blist2[pl.ds(t * L, L)], mask=m)

            def lab_body(l, fired):
                lsel = jnp.full((L,), l, jnp.int32)
                r = plsc.load_gather(s16r, [lsel])[0]
                b = plsc.load_gather(s16b, [lsel])[0]
                cl = jnp.full((L,), r & 127, jnp.int32)
                v0 = plsc.load_gather(buf, [iota, cl])
                v1 = plsc.load_gather(buf, [iota + L, cl])
                v2 = plsc.load_gather(buf, [iota + 2 * L, cl])
                v3 = plsc.load_gather(buf, [iota + 3 * L, cl])
                acc = v0 * v0 + v1 * v1 + v2 * v2 + v3 * v3
                for sh in (8, 4, 2, 1):
                    acc = acc + _lane_shuffle(acc, iota ^ sh)
                x = acc
                i = plsc.bitcast(x, jnp.int32)
                y = plsc.bitcast(jnp.int32(0x5F3759DF) - (i >> 1), jnp.float32)
                h = x * 0.5
                for _ in range(3):
                    y = y * (1.5 - (h * y) * y)
                norm = x * y
                scale = jnp.minimum(1.0, 1.0 / (norm + 1e-7))
                slot = fired & (RING - 1)
                pl.when(fired >= RING)(
                    lambda: pltpu.make_async_copy(
                        dummy_row, ring.at[slot], sem_out
                    ).wait()
                )
                ring[slot, pl.ds(0, L)] = v0 * scale
                ring[slot, pl.ds(L, L)] = v1 * scale
                ring[slot, pl.ds(2 * L, L)] = v2 * scale
                ring[slot, pl.ds(3 * L, L)] = v3 * scale
                pltpu.async_copy(
                    ring.at[slot], out_hbm.at[pl.ds(b * D, D)], sem_out
                )
                return fired + 1

            return lax.fori_loop(0, nm, lab_body, fired)

        return lax.fori_loop(t0, t1, scan_body, fired)

    fired = lax.fori_loop(0, ntiles, tile_body, jnp.int32(0))

    def absorb_body(p, carry):
        pltpu.make_async_copy(
            tableT_hbm.at[:, pl.ds(0, 128)],
            tiles.at[(ntiles + p) & (NBUF - 1)],
            sem_in,
        ).wait()
        return carry

    lax.fori_loop(0, NBUF - 1, absorb_body, 0)

    def drain_body(d, carry):
        pl.when(d < jnp.minimum(fired, RING))(
            lambda: pltpu.make_async_copy(
                dummy_row, ring.at[0], sem_out
            ).wait()
        )
        return carry

    lax.fori_loop(0, RING, drain_body, 0)


def kernel(table, labels):
    out = _embed(table.T, labels.astype(jnp.int32))
    return out.reshape(B, D)
